# baseline (speedup 1.0000x reference)
import functools

import jax
import jax.numpy as jnp
from jax import lax
from jax.experimental import pallas as pl
from jax.experimental.pallas import tpu as pltpu
from jax.experimental.pallas import tpu_sc as plsc

N_PTS = 262144
C = 32
RES = (64, 128, 256, 512)
COMBS = ((0, 1), (0, 2), (1, 2))
L = 16
NW = 32
PTS_PER_W = N_PTS // NW
B = 128
NBLK = PTS_PER_W // B

_sc_mesh = plsc.VectorSubcoreMesh(core_axis_name="c", subcore_axis_name="s",
                                  num_cores=2, num_subcores=16)


_BV = 1024


def _pack_body(*refs):
    for g_ref, o_ref in zip(refs[0:3], refs[3:6]):
        o_ref[...] = g_ref[...].T


def _pack_tables3(g0, g1, g2):
    v = g0.shape[1]
    cur_spec = pl.BlockSpec((C, _BV), lambda i: (0, i))
    out_spec = pl.BlockSpec((_BV, C), lambda i: (i, 0))
    return pl.pallas_call(
        _pack_body,
        grid=(v // _BV,),
        in_specs=[cur_spec] * 3,
        out_specs=[out_spec] * 3,
        out_shape=[jax.ShapeDtypeStruct((v, C), jnp.float32)] * 3,
    )(g0, g1, g2)


@functools.partial(
    pl.kernel,
    out_type=jax.ShapeDtypeStruct((N_PTS, 4 * C), jnp.float32),
    mesh=_sc_mesh,
    scratch_types=[
        pltpu.VMEM((3, B), jnp.float32),
        pltpu.VMEM((2, 3, 4, B), jnp.int32),
        pltpu.VMEM((2, 3, 2, B + L), jnp.float32),
        pltpu.VMEM((2, 3, 4, B, C), jnp.float32),
        pltpu.VMEM((B, 4 * C), jnp.float32),
        pltpu.SemaphoreType.DMA,
        pltpu.SemaphoreType.DMA,
    ],
    compiler_params=pltpu.CompilerParams(use_tc_tiling_on_sc=False),
)
def _sc_encode(xh,
               t00, t01, t02, t10, t11, t12, t20, t21, t22, t30, t31, t32,
               featsh, xv, idxv, wv, gb, fb, sem0, sem1):
    tables = (t00, t01, t02, t10, t11, t12, t20, t21, t22, t30, t31, t32)
    sems = (sem0, sem1)
    wid = lax.axis_index("s") * 2 + lax.axis_index("c")
    base0 = wid * PTS_PER_W

    def load_x(base):
        for d in range(3):
            pltpu.sync_copy(xh.at[d, pl.ds(base, B)], xv.at[d])

    def compute_idx(s, slot):
        R = RES[s]

        @plsc.parallel_loop(0, B // L)
        def idx_body(g):
            sl = pl.ds(g * L, L)
            iis = []
            ws = []
            for d in range(3):
                f = (xv[d, sl] + 1.0) * (0.5 * (R - 1))
                i = jnp.minimum(f.astype(jnp.int32), R - 2)
                iis.append(i)
                ws.append(f - i.astype(jnp.float32))
            for p, (a, b) in enumerate(COMBS):
                bi = iis[b] * R + iis[a]
                idxv[slot, p, 0, sl] = bi
                idxv[slot, p, 1, sl] = bi + 1
                idxv[slot, p, 2, sl] = bi + R
                idxv[slot, p, 3, sl] = bi + (R + 1)
                wv[slot, p, 0, sl] = ws[a]
                wv[slot, p, 1, sl] = ws[b]

    def fire(s, slot):
        for p in range(3):
            for k in range(4):
                pltpu.async_copy(tables[s * 3 + p].at[idxv.at[slot, p, k]],
                                 gb.at[slot, p, k], sems[slot])

    def drain(slot):
        for _ in range(12):
            pltpu.make_async_copy(tables[0].at[pl.ds(0, B)],
                                  gb.at[slot, 0, 0], sems[slot]).wait()

    def combine(s, slot):
        @plsc.parallel_loop(0, B // L, unroll=2)
        def comb_body(g):
            gsl = pl.ds(g * L, L)
            wvecs = [(wv[slot, p, 0, gsl], wv[slot, p, 1, gsl])
                     for p in range(3)]
            for jj in range(L):
                j = g * L + jj
                fe = None
                fo = None
                for p in range(3):
                    wx = wvecs[p][0][jj]
                    wy = wvecs[p][1][jj]
                    for h in range(2):
                        hsl = pl.ds(h * L, L)
                        g00 = gb[slot, p, 0, j, hsl]
                        g01 = gb[slot, p, 1, j, hsl]
                        g10 = gb[slot, p, 2, j, hsl]
                        g11 = gb[slot, p, 3, j, hsl]
                        gx0 = g00 + (g01 - g00) * wx
                        gx1 = g10 + (g11 - g10) * wx
                        v = gx0 + (gx1 - gx0) * wy
                        if h == 0:
                            fe = v if fe is None else fe * v
                        else:
                            fo = v if fo is None else fo * v
                fb[j, pl.ds(s * C, L)] = fe
                fb[j, pl.ds(s * C + L, L)] = fo

    load_x(base0)
    compute_idx(0, 0)
    fire(0, 0)

    def block_body(blk, carry):
        base = base0 + blk * B
        for s in range(4):
            if s < 3:
                compute_idx(s + 1, (s + 1) % 2)
                fire(s + 1, (s + 1) % 2)
            else:
                @pl.when(blk < NBLK - 1)
                def _prefire():
                    load_x(base + B)
                    compute_idx(0, 0)
                    fire(0, 0)
            drain(s % 2)
            combine(s, s % 2)
        pltpu.sync_copy(fb, featsh.at[pl.ds(base, B)])
        return carry

    lax.fori_loop(0, NBLK, block_body, 0)


BN = 8192


def _mlp_body(f_ref, w1_ref, w2_ref, w3_ref, o_ref):
    h = jnp.maximum(jnp.dot(f_ref[...], w1_ref[...],
                            preferred_element_type=jnp.float32), 0.0)
    h = jnp.maximum(jnp.dot(h, w2_ref[...],
                            preferred_element_type=jnp.float32), 0.0)
    o_ref[...] = jnp.dot(h, w3_ref[...], preferred_element_type=jnp.float32)


_mlp = pl.pallas_call(
    _mlp_body,
    grid=(N_PTS // BN,),
    in_specs=[
        pl.BlockSpec((BN, 4 * C), lambda i: (i, 0)),
        pl.BlockSpec((4 * C, 64), lambda i: (0, 0)),
        pl.BlockSpec((64, 64), lambda i: (0, 0)),
        pl.BlockSpec((64, 1), lambda i: (0, 0)),
    ],
    out_specs=pl.BlockSpec((BN, 1), lambda i: (i, 0)),
    out_shape=jax.ShapeDtypeStruct((N_PTS, 1), jnp.float32),
)


def kernel(x, grid_0_0, grid_0_1, grid_0_2, grid_1_0, grid_1_1, grid_1_2,
           grid_2_0, grid_2_1, grid_2_2, grid_3_0, grid_3_1, grid_3_2,
           W1, W2, W3):
    grids = (grid_0_0, grid_0_1, grid_0_2, grid_1_0, grid_1_1, grid_1_2,
             grid_2_0, grid_2_1, grid_2_2, grid_3_0, grid_3_1, grid_3_2)
    tables = []
    for s in range(4):
        tables.extend(_pack_tables3(*(g.reshape(C, -1)
                                      for g in grids[3 * s:3 * s + 3])))
    xt = jnp.transpose(x)
    feats = _sc_encode(xt, *tables)
    return _mlp(feats, W1, W2, W3)


# speedup vs baseline: 1.5400x; 1.5400x over previous
import functools

import jax
import jax.numpy as jnp
from jax import lax
from jax.experimental import pallas as pl
from jax.experimental.pallas import tpu as pltpu
from jax.experimental.pallas import tpu_sc as plsc

N_PTS = 262144
C = 32
RES = (64, 128, 256, 512)
COMBS = ((0, 1), (0, 2), (1, 2))
L = 16
NW = 32
PTS_PER_W = N_PTS // NW
B = 128
NBLK = PTS_PER_W // B

_sc_mesh = plsc.VectorSubcoreMesh(core_axis_name="c", subcore_axis_name="s",
                                  num_cores=2, num_subcores=16)


_BV = 1024


def _pack_body(*refs):
    for g_ref, o_ref in zip(refs[0:3], refs[3:6]):
        o_ref[...] = g_ref[...].T


def _pack_tables3(g0, g1, g2):
    v = g0.shape[1]
    cur_spec = pl.BlockSpec((C, _BV), lambda i: (0, i))
    out_spec = pl.BlockSpec((_BV, C), lambda i: (i, 0))
    return pl.pallas_call(
        _pack_body,
        grid=(v // _BV,),
        in_specs=[cur_spec] * 3,
        out_specs=[out_spec] * 3,
        out_shape=[jax.ShapeDtypeStruct((v, C), jnp.float32)] * 3,
    )(g0, g1, g2)


@functools.partial(
    pl.kernel,
    out_type=jax.ShapeDtypeStruct((N_PTS, 4 * C), jnp.float32),
    mesh=_sc_mesh,
    scratch_types=[
        pltpu.VMEM((3, B), jnp.float32),
        pltpu.VMEM((2, 3, 4, B), jnp.int32),
        pltpu.VMEM((2, 3, 2, B + L), jnp.float32),
        pltpu.VMEM((2, 3, 4, B, C), jnp.float32),
        pltpu.VMEM((B, 4 * C), jnp.float32),
        pltpu.SemaphoreType.DMA,
        pltpu.SemaphoreType.DMA,
    ],
    compiler_params=pltpu.CompilerParams(use_tc_tiling_on_sc=False),
)
def _sc_encode(xh,
               t00, t01, t02, t10, t11, t12, t20, t21, t22, t30, t31, t32,
               featsh, xv, idxv, wv, gb, fb, sem0, sem1):
    tables = (t00, t01, t02, t10, t11, t12, t20, t21, t22, t30, t31, t32)
    sems = (sem0, sem1)
    wid = lax.axis_index("s") * 2 + lax.axis_index("c")
    base0 = wid * PTS_PER_W

    def load_x(base):
        for d in range(3):
            pltpu.sync_copy(xh.at[d, pl.ds(base, B)], xv.at[d])

    def compute_idx(s, slot):
        R = RES[s]

        @plsc.parallel_loop(0, B // L)
        def idx_body(g):
            sl = pl.ds(g * L, L)
            iis = []
            ws = []
            for d in range(3):
                f = (xv[d, sl] + 1.0) * (0.5 * (R - 1))
                i = jnp.minimum(f.astype(jnp.int32), R - 2)
                iis.append(i)
                ws.append(f - i.astype(jnp.float32))
            for p, (a, b) in enumerate(COMBS):
                bi = iis[b] * R + iis[a]
                idxv[slot, p, 0, sl] = bi
                idxv[slot, p, 1, sl] = bi + 1
                idxv[slot, p, 2, sl] = bi + R
                idxv[slot, p, 3, sl] = bi + (R + 1)
                wv[slot, p, 0, sl] = ws[a]
                wv[slot, p, 1, sl] = ws[b]

    def fire(s, slot):
        for p in range(3):
            for k in range(4):
                pltpu.async_copy(tables[s * 3 + p].at[idxv.at[slot, p, k]],
                                 gb.at[slot, p, k], sems[slot])

    def drain(slot):
        for _ in range(12):
            pltpu.make_async_copy(tables[0].at[pl.ds(0, B)],
                                  gb.at[slot, 0, 0], sems[slot]).wait()

    def combine(s, slot):
        @plsc.parallel_loop(0, B // L)
        def comb_body(g):
            gsl = pl.ds(g * L, L)
            wvecs = [(wv[slot, p, 0, gsl], wv[slot, p, 1, gsl])
                     for p in range(3)]
            for jj in range(L):
                j = g * L + jj
                fe = None
                fo = None
                for p in range(3):
                    wx = wvecs[p][0][jj]
                    wy = wvecs[p][1][jj]
                    for h in range(2):
                        hsl = pl.ds(h * L, L)
                        g00 = gb[slot, p, 0, j, hsl]
                        g01 = gb[slot, p, 1, j, hsl]
                        g10 = gb[slot, p, 2, j, hsl]
                        g11 = gb[slot, p, 3, j, hsl]
                        gx0 = g00 + (g01 - g00) * wx
                        gx1 = g10 + (g11 - g10) * wx
                        v = gx0 + (gx1 - gx0) * wy
                        if h == 0:
                            fe = v if fe is None else fe * v
                        else:
                            fo = v if fo is None else fo * v
                fb[j, pl.ds(s * C, L)] = fe
                fb[j, pl.ds(s * C + L, L)] = fo

    load_x(base0)
    compute_idx(0, 0)
    fire(0, 0)

    def block_body(blk, carry):
        base = base0 + blk * B
        for s in range(4):
            if s < 3:
                compute_idx(s + 1, (s + 1) % 2)
                fire(s + 1, (s + 1) % 2)
            else:
                @pl.when(blk < NBLK - 1)
                def _prefire():
                    load_x(base + B)
                    compute_idx(0, 0)
                    fire(0, 0)
            drain(s % 2)
            combine(s, s % 2)
        pltpu.sync_copy(fb, featsh.at[pl.ds(base, B)])
        return carry

    lax.fori_loop(0, NBLK, block_body, 0)


BN = 2048


def _mlp_body(f_ref, w1_ref, w2_ref, w3_ref, o_ref):
    h = jnp.maximum(jnp.dot(f_ref[...], w1_ref[...],
                            preferred_element_type=jnp.float32), 0.0)
    h = jnp.maximum(jnp.dot(h, w2_ref[...],
                            preferred_element_type=jnp.float32), 0.0)
    o_ref[...] = jnp.dot(h, w3_ref[...], preferred_element_type=jnp.float32)


_mlp = pl.pallas_call(
    _mlp_body,
    grid=(N_PTS // BN,),
    in_specs=[
        pl.BlockSpec((BN, 4 * C), lambda i: (i, 0)),
        pl.BlockSpec((4 * C, 64), lambda i: (0, 0)),
        pl.BlockSpec((64, 64), lambda i: (0, 0)),
        pl.BlockSpec((64, 1), lambda i: (0, 0)),
    ],
    out_specs=pl.BlockSpec((BN, 1), lambda i: (i, 0)),
    out_shape=jax.ShapeDtypeStruct((N_PTS, 1), jnp.float32),
)


def kernel(x, grid_0_0, grid_0_1, grid_0_2, grid_1_0, grid_1_1, grid_1_2,
           grid_2_0, grid_2_1, grid_2_2, grid_3_0, grid_3_1, grid_3_2,
           W1, W2, W3):
    grids = (grid_0_0, grid_0_1, grid_0_2, grid_1_0, grid_1_1, grid_1_2,
             grid_2_0, grid_2_1, grid_2_2, grid_3_0, grid_3_1, grid_3_2)
    tables = []
    for s in range(4):
        tables.extend(_pack_tables3(*(g.reshape(C, -1)
                                      for g in grids[3 * s:3 * s + 3])))
    xt = jnp.transpose(x)
    feats = _sc_encode(xt, *tables)
    return _mlp(feats, W1, W2, W3)
